# baseline (speedup 1.0000x reference)
import jax
import jax.numpy as jnp
from jax.experimental import pallas as pl
from jax.experimental.pallas import tpu as pltpu

_B = 16
_C = 128
_L = 2048
_D2 = 256
_TK = 3
_N = _B * _L
_ROWS = _L
_EPS = 1e-5
_LC = _L // _C
_FB = 4
_MB = 4
_TB = 4

_CONTRACT_R1 = (((1,), (1,)), ((), ()))


def _bn0_coeffs(s_ref, q_ref, g_ref, b_ref):
    mu = s_ref[...] * (1.0 / (_B * _L))
    var = q_ref[...] * (1.0 / (_B * _L)) - mu * mu
    sc = jnp.swapaxes(g_ref[...], 0, 1) * jax.lax.rsqrt(var + _EPS)
    sh = jnp.swapaxes(b_ref[...], 0, 1) - mu * sc
    return sc, sh


def _bn_coeffs(sp_ref, qp_ref, g_ref, b_ref):
    s = jnp.sum(sp_ref[...], axis=(0, 1))[None, :]
    q = jnp.sum(qp_ref[...], axis=(0, 1))[None, :]
    mu = s * (1.0 / _N)
    var = q * (1.0 / _N) - mu * mu
    sc = g_ref[...] * jax.lax.rsqrt(var + _EPS)
    sh = b_ref[...] - mu * sc
    return sc, sh


def _front_kernel(x_ref, w1_ref, u_ref, s_ref, q_ref, u1_ref, u2_ref):
    b = pl.program_id(0)
    xb = x_ref[...]
    s = jnp.sum(xb, axis=(0, 2))[:, None]
    q = jnp.sum(xb * xb, axis=(0, 2))[:, None]
    w1b = w1_ref[...].astype(jnp.bfloat16)
    uu1 = jnp.zeros((_C, _D2), jnp.float32)
    uu2 = jnp.zeros((_C, _D2), jnp.float32)
    for t in range(_FB):
        hf = jnp.reshape(xb[t], (_ROWS, _C))
        u = jax.lax.dot_general(hf.astype(jnp.bfloat16), w1b,
                                _CONTRACT_R1,
                                preferred_element_type=jnp.float32)
        ub = u.astype(jnp.bfloat16)
        u_ref[t * _ROWS:(t + 1) * _ROWS, :] = ub
        uf = ub.astype(jnp.float32)
        u4 = jnp.reshape(uf, (_C, _LC, _D2))
        uu1 = uu1 + jnp.sum(u4, axis=1)
        uu2 = uu2 + jnp.sum(u4 * u4, axis=1)

    @pl.when(b == 0)
    def _init():
        s_ref[...] = s
        q_ref[...] = q
        u1_ref[...] = uu1
        u2_ref[...] = uu2

    @pl.when(b > 0)
    def _acc():
        s_ref[...] = s_ref[...] + s
        q_ref[...] = q_ref[...] + q
        u1_ref[...] = u1_ref[...] + uu1
        u2_ref[...] = u2_ref[...] + uu2


def _mid1_kernel(u_ref, s0_ref, q0_ref, g0_ref, b0_ref, u1_ref, u2_ref,
                 w1_ref, b1_ref, g_ref, b_ref, w_ref, bias_ref,
                 o_ref, sp_ref, qp_ref):
    a, cc = _bn0_coeffs(s0_ref, q0_ref, g0_ref, b0_ref)
    w1s = jnp.swapaxes(jnp.sum(w1_ref[...], axis=1, keepdims=True), 0, 1)
    d = cc * w1s + b1_ref[...]
    rows_per_c = _B * _LC
    s1 = jnp.sum(a * u1_ref[...], axis=0, keepdims=True)         + rows_per_c * jnp.sum(d, axis=0, keepdims=True)
    q1 = jnp.sum(a * a * u2_ref[...] + 2.0 * a * d * u1_ref[...]
                 + rows_per_c * d * d, axis=0, keepdims=True)
    mu = s1 * (1.0 / _N)
    var = q1 * (1.0 / _N) - mu * mu
    sc1 = g_ref[...] * jax.lax.rsqrt(var + _EPS)
    sh1 = b_ref[...] - mu * sc1
    aa = (a * sc1)[None, :, None, :]
    bb = (d * sc1 + sh1)[None, :, None, :]
    u4 = jnp.reshape(u_ref[...].astype(jnp.float32), (_MB, _C, _LC, _D2))
    zz4 = jnp.maximum(u4 * aa + bb, 0.0)
    zz = jnp.reshape(zz4, (_MB * _ROWS, _D2))
    y2 = jax.lax.dot_general(zz.astype(jnp.bfloat16),
                             w_ref[...].astype(jnp.bfloat16),
                             _CONTRACT_R1,
                             preferred_element_type=jnp.float32)
    y2 = y2 + bias_ref[...]
    o_ref[...] = y2.astype(jnp.bfloat16)
    sp_ref[...] = jnp.sum(y2, axis=0, keepdims=True)[None]
    qp_ref[...] = jnp.sum(y2 * y2, axis=0, keepdims=True)[None]


def _mid_kernel(y_ref, spi_ref, qpi_ref, g_ref, b_ref, w_ref, bias_ref,
                o_ref, sp_ref, qp_ref):
    sc, sh = _bn_coeffs(spi_ref, qpi_ref, g_ref, b_ref)
    yv = y_ref[...].astype(jnp.float32)
    zz = jnp.maximum(yv * sc + sh, 0.0)
    y2 = jax.lax.dot_general(zz.astype(jnp.bfloat16),
                             w_ref[...].astype(jnp.bfloat16),
                             _CONTRACT_R1,
                             preferred_element_type=jnp.float32)
    y2 = y2 + bias_ref[...]
    o_ref[...] = y2.astype(jnp.bfloat16)
    sp_ref[...] = jnp.sum(y2, axis=0, keepdims=True)[None]
    qp_ref[...] = jnp.sum(y2 * y2, axis=0, keepdims=True)[None]


def _tail_kernel(x_ref, s0_ref, q0_ref, g0_ref, b0_ref, wc_ref, bsk_ref,
                 y_ref, spi_ref, qpi_ref, g1_ref, b1_ref, o_ref):
    scc, shc = _bn0_coeffs(s0_ref, q0_ref, g0_ref, b0_ref)
    sc, sh = _bn_coeffs(spi_ref, qpi_ref, g1_ref, b1_ref)
    for t in range(_TB):
        xn = x_ref[t] * scc + shc
        z = jnp.zeros((_C, 1), jnp.float32)
        xm1 = jnp.concatenate([z, xn[:, :-1]], axis=1)
        xp1 = jnp.concatenate([xn[:, 1:], z], axis=1)
        xcat = jnp.concatenate([xm1, xn, xp1], axis=0)
        r = jnp.dot(wc_ref[...], xcat.astype(jnp.bfloat16),
                    preferred_element_type=jnp.float32)
        resb = jnp.maximum(r + jnp.swapaxes(bsk_ref[...], 0, 1), 0.0)
        yv = y_ref[t * _ROWS:(t + 1) * _ROWS, :].astype(jnp.float32)
        zz = jnp.maximum(yv * sc + sh, 0.0)
        o_ref[t] = resb + jnp.reshape(zz, (_D2, _L))


def kernel(x, edge_index, train, gamma0, beta0, Wskip, bskip, W1, bias1,
           gamma1, beta1, W2, bias2, W3, bias3):
    del edge_index, train
    f32 = jnp.float32
    bf16 = jnp.bfloat16

    g0c = gamma0.reshape(1, _C)
    b0c = beta0.reshape(1, _C)
    g1r = gamma1.reshape(1, _D2)
    b1r = beta1.reshape(1, _D2)

    wc = jnp.concatenate([Wskip[:, :, 0], Wskip[:, :, 1], Wskip[:, :, 2]],
                         axis=1).astype(bf16)
    bsk = bskip.reshape(1, _D2)

    _vec = lambda b: (0, 0)
    _vec3 = lambda b: (0, 0, 0)

    u, s0, q0, U1, U2 = pl.pallas_call(
        _front_kernel,
        grid=(_B // _FB,),
        in_specs=[
            pl.BlockSpec((_FB, _C, _L), lambda b: (b, 0, 0)),
            pl.BlockSpec((_D2, _C), _vec),
        ],
        out_specs=[
            pl.BlockSpec((_FB * _ROWS, _D2), lambda b: (b, 0)),
            pl.BlockSpec((_C, 1), _vec),
            pl.BlockSpec((_C, 1), _vec),
            pl.BlockSpec((_C, _D2), _vec),
            pl.BlockSpec((_C, _D2), _vec),
        ],
        out_shape=[
            jax.ShapeDtypeStruct((_N, _D2), bf16),
            jax.ShapeDtypeStruct((_C, 1), f32),
            jax.ShapeDtypeStruct((_C, 1), f32),
            jax.ShapeDtypeStruct((_C, _D2), f32),
            jax.ShapeDtypeStruct((_C, _D2), f32),
        ],
    )(x, W1)

    nm = _B // _MB
    y2, s2p, q2p = pl.pallas_call(
        _mid1_kernel,
        grid=(nm,),
        in_specs=[
            pl.BlockSpec((_MB * _ROWS, _D2), lambda b: (b, 0)),
            pl.BlockSpec((_C, 1), _vec),
            pl.BlockSpec((_C, 1), _vec),
            pl.BlockSpec((1, _C), _vec),
            pl.BlockSpec((1, _C), _vec),
            pl.BlockSpec((_C, _D2), _vec),
            pl.BlockSpec((_C, _D2), _vec),
            pl.BlockSpec((_D2, _C), _vec),
            pl.BlockSpec((1, _D2), _vec),
            pl.BlockSpec((1, _D2), _vec),
            pl.BlockSpec((1, _D2), _vec),
            pl.BlockSpec((_D2, _D2), _vec),
            pl.BlockSpec((1, _D2), _vec),
        ],
        out_specs=[
            pl.BlockSpec((_MB * _ROWS, _D2), lambda b: (b, 0)),
            pl.BlockSpec((1, 1, _D2), lambda b: (b, 0, 0)),
            pl.BlockSpec((1, 1, _D2), lambda b: (b, 0, 0)),
        ],
        out_shape=[
            jax.ShapeDtypeStruct((_N, _D2), bf16),
            jax.ShapeDtypeStruct((nm, 1, _D2), f32),
            jax.ShapeDtypeStruct((nm, 1, _D2), f32),
        ],
    )(u, s0, q0, g0c, b0c, U1, U2, W1, bias1.reshape(1, _D2),
      g1r, b1r, W2, bias2.reshape(1, _D2))

    def mid(y, sp, qp, w, bias):
        nm = _B // _MB
        return pl.pallas_call(
            _mid_kernel,
            grid=(nm,),
            in_specs=[
                pl.BlockSpec((_MB * _ROWS, _D2), lambda b: (b, 0)),
                pl.BlockSpec(sp.shape, _vec3),
                pl.BlockSpec(qp.shape, _vec3),
                pl.BlockSpec((1, _D2), _vec),
                pl.BlockSpec((1, _D2), _vec),
                pl.BlockSpec((_D2, _D2), _vec),
                pl.BlockSpec((1, _D2), _vec),
            ],
            out_specs=[
                pl.BlockSpec((_MB * _ROWS, _D2), lambda b: (b, 0)),
                pl.BlockSpec((1, 1, _D2), lambda b: (b, 0, 0)),
                pl.BlockSpec((1, 1, _D2), lambda b: (b, 0, 0)),
            ],
            out_shape=[
                jax.ShapeDtypeStruct((_N, _D2), bf16),
                jax.ShapeDtypeStruct((nm, 1, _D2), f32),
                jax.ShapeDtypeStruct((nm, 1, _D2), f32),
            ],
        )(y, sp, qp, g1r, b1r, w, bias.reshape(1, _D2))


    y3, s3p, q3p = mid(y2, s2p, q2p, W3, bias3)

    out = pl.pallas_call(
        _tail_kernel,
        grid=(_B // _TB,),
        in_specs=[
            pl.BlockSpec((_TB, _C, _L), lambda b: (b, 0, 0)),
            pl.BlockSpec((_C, 1), _vec),
            pl.BlockSpec((_C, 1), _vec),
            pl.BlockSpec((1, _C), _vec),
            pl.BlockSpec((1, _C), _vec),
            pl.BlockSpec((_D2, _TK * _C), _vec),
            pl.BlockSpec((1, _D2), _vec),
            pl.BlockSpec((_TB * _ROWS, _D2), lambda b: (b, 0)),
            pl.BlockSpec(s3p.shape, _vec3),
            pl.BlockSpec(q3p.shape, _vec3),
            pl.BlockSpec((1, _D2), _vec),
            pl.BlockSpec((1, _D2), _vec),
        ],
        out_specs=pl.BlockSpec((_TB, _D2, _L), lambda b: (b, 0, 0)),
        out_shape=jax.ShapeDtypeStruct((_B, _D2, _L), f32),
    )(x, s0, q0, g0c, b0c, wc, bsk, y3, s3p, q3p, g1r, b1r)

    return out


# speedup vs baseline: 1.0377x; 1.0377x over previous
import jax
import jax.numpy as jnp
from jax.experimental import pallas as pl
from jax.experimental.pallas import tpu as pltpu

_B = 16
_C = 128
_L = 2048
_D2 = 256
_TK = 3
_N = _B * _L
_ROWS = _L
_EPS = 1e-5
_LC = _L // _C
_FB = 4
_MB = 4
_TB = 4

_CONTRACT_R1 = (((1,), (1,)), ((), ()))


def _bn0_coeffs(s_ref, q_ref, g_ref, b_ref):
    mu = s_ref[...] * (1.0 / (_B * _L))
    var = q_ref[...] * (1.0 / (_B * _L)) - mu * mu
    sc = jnp.swapaxes(g_ref[...], 0, 1) * jax.lax.rsqrt(var + _EPS)
    sh = jnp.swapaxes(b_ref[...], 0, 1) - mu * sc
    return sc, sh


def _bn_coeffs(sp_ref, qp_ref, g_ref, b_ref):
    s = jnp.sum(sp_ref[...], axis=(0, 1))[None, :]
    q = jnp.sum(qp_ref[...], axis=(0, 1))[None, :]
    mu = s * (1.0 / _N)
    var = q * (1.0 / _N) - mu * mu
    sc = g_ref[...] * jax.lax.rsqrt(var + _EPS)
    sh = b_ref[...] - mu * sc
    return sc, sh


def _front_kernel(x_ref, w1_ref, u_ref, s_ref, q_ref, u1_ref, u2_ref):
    b = pl.program_id(0)
    xb = x_ref[...]
    s = jnp.sum(xb, axis=(0, 2))[:, None]
    q = jnp.sum(xb * xb, axis=(0, 2))[:, None]
    w1b = w1_ref[...].astype(jnp.bfloat16)
    smat = (jax.lax.broadcasted_iota(jnp.int32, (_C, _ROWS), 0)
            == jax.lax.broadcasted_iota(jnp.int32, (_C, _ROWS), 1) // _LC
            ).astype(jnp.bfloat16)
    uu1 = jnp.zeros((_C, _D2), jnp.float32)
    uu2 = jnp.zeros((_C, _D2), jnp.float32)
    for t in range(_FB):
        hf = jnp.reshape(xb[t], (_ROWS, _C))
        u = jax.lax.dot_general(hf.astype(jnp.bfloat16), w1b,
                                _CONTRACT_R1,
                                preferred_element_type=jnp.float32)
        ub = u.astype(jnp.bfloat16)
        u_ref[t * _ROWS:(t + 1) * _ROWS, :] = ub
        u2b = (u * u).astype(jnp.bfloat16)
        uu1 = uu1 + jnp.dot(smat, ub, preferred_element_type=jnp.float32)
        uu2 = uu2 + jnp.dot(smat, u2b, preferred_element_type=jnp.float32)

    @pl.when(b == 0)
    def _init():
        s_ref[...] = s
        q_ref[...] = q
        u1_ref[...] = uu1
        u2_ref[...] = uu2

    @pl.when(b > 0)
    def _acc():
        s_ref[...] = s_ref[...] + s
        q_ref[...] = q_ref[...] + q
        u1_ref[...] = u1_ref[...] + uu1
        u2_ref[...] = u2_ref[...] + uu2


def _mid1_kernel(u_ref, s0_ref, q0_ref, g0_ref, b0_ref, u1_ref, u2_ref,
                 w1_ref, b1_ref, g_ref, b_ref, w_ref, bias_ref,
                 o_ref, sp_ref, qp_ref):
    a, cc = _bn0_coeffs(s0_ref, q0_ref, g0_ref, b0_ref)
    w1s = jnp.swapaxes(jnp.sum(w1_ref[...], axis=1, keepdims=True), 0, 1)
    d = cc * w1s + b1_ref[...]
    rows_per_c = _B * _LC
    s1 = jnp.sum(a * u1_ref[...], axis=0, keepdims=True)         + rows_per_c * jnp.sum(d, axis=0, keepdims=True)
    q1 = jnp.sum(a * a * u2_ref[...] + 2.0 * a * d * u1_ref[...]
                 + rows_per_c * d * d, axis=0, keepdims=True)
    mu = s1 * (1.0 / _N)
    var = q1 * (1.0 / _N) - mu * mu
    sc1 = g_ref[...] * jax.lax.rsqrt(var + _EPS)
    sh1 = b_ref[...] - mu * sc1
    aa = (a * sc1)[None, :, None, :]
    bb = (d * sc1 + sh1)[None, :, None, :]
    u4 = jnp.reshape(u_ref[...].astype(jnp.float32), (_MB, _C, _LC, _D2))
    zz4 = jnp.maximum(u4 * aa + bb, 0.0)
    zz = jnp.reshape(zz4, (_MB * _ROWS, _D2))
    y2 = jax.lax.dot_general(zz.astype(jnp.bfloat16),
                             w_ref[...].astype(jnp.bfloat16),
                             _CONTRACT_R1,
                             preferred_element_type=jnp.float32)
    y2 = y2 + bias_ref[...]
    o_ref[...] = y2.astype(jnp.bfloat16)
    sp_ref[...] = jnp.sum(y2, axis=0, keepdims=True)[None]
    qp_ref[...] = jnp.sum(y2 * y2, axis=0, keepdims=True)[None]


def _mid_kernel(y_ref, spi_ref, qpi_ref, g_ref, b_ref, w_ref, bias_ref,
                o_ref, sp_ref, qp_ref):
    sc, sh = _bn_coeffs(spi_ref, qpi_ref, g_ref, b_ref)
    yv = y_ref[...].astype(jnp.float32)
    zz = jnp.maximum(yv * sc + sh, 0.0)
    y2 = jax.lax.dot_general(zz.astype(jnp.bfloat16),
                             w_ref[...].astype(jnp.bfloat16),
                             _CONTRACT_R1,
                             preferred_element_type=jnp.float32)
    y2 = y2 + bias_ref[...]
    o_ref[...] = y2.astype(jnp.bfloat16)
    sp_ref[...] = jnp.sum(y2, axis=0, keepdims=True)[None]
    qp_ref[...] = jnp.sum(y2 * y2, axis=0, keepdims=True)[None]


def _tail_kernel(x_ref, s0_ref, q0_ref, g0_ref, b0_ref, wc_ref, bsk_ref,
                 y_ref, spi_ref, qpi_ref, g1_ref, b1_ref, o_ref):
    scc, shc = _bn0_coeffs(s0_ref, q0_ref, g0_ref, b0_ref)
    sc, sh = _bn_coeffs(spi_ref, qpi_ref, g1_ref, b1_ref)
    for t in range(_TB):
        xn = x_ref[t] * scc + shc
        z = jnp.zeros((_C, 1), jnp.float32)
        xm1 = jnp.concatenate([z, xn[:, :-1]], axis=1)
        xp1 = jnp.concatenate([xn[:, 1:], z], axis=1)
        xcat = jnp.concatenate([xm1, xn, xp1], axis=0)
        r = jnp.dot(wc_ref[...], xcat.astype(jnp.bfloat16),
                    preferred_element_type=jnp.float32)
        resb = jnp.maximum(r + jnp.swapaxes(bsk_ref[...], 0, 1), 0.0)
        yv = y_ref[t * _ROWS:(t + 1) * _ROWS, :].astype(jnp.float32)
        zz = jnp.maximum(yv * sc + sh, 0.0)
        o_ref[t] = resb + jnp.reshape(zz, (_D2, _L))


def kernel(x, edge_index, train, gamma0, beta0, Wskip, bskip, W1, bias1,
           gamma1, beta1, W2, bias2, W3, bias3):
    del edge_index, train
    f32 = jnp.float32
    bf16 = jnp.bfloat16

    g0c = gamma0.reshape(1, _C)
    b0c = beta0.reshape(1, _C)
    g1r = gamma1.reshape(1, _D2)
    b1r = beta1.reshape(1, _D2)

    wc = jnp.concatenate([Wskip[:, :, 0], Wskip[:, :, 1], Wskip[:, :, 2]],
                         axis=1).astype(bf16)
    bsk = bskip.reshape(1, _D2)

    _vec = lambda b: (0, 0)
    _vec3 = lambda b: (0, 0, 0)

    u, s0, q0, U1, U2 = pl.pallas_call(
        _front_kernel,
        grid=(_B // _FB,),
        in_specs=[
            pl.BlockSpec((_FB, _C, _L), lambda b: (b, 0, 0)),
            pl.BlockSpec((_D2, _C), _vec),
        ],
        out_specs=[
            pl.BlockSpec((_FB * _ROWS, _D2), lambda b: (b, 0)),
            pl.BlockSpec((_C, 1), _vec),
            pl.BlockSpec((_C, 1), _vec),
            pl.BlockSpec((_C, _D2), _vec),
            pl.BlockSpec((_C, _D2), _vec),
        ],
        out_shape=[
            jax.ShapeDtypeStruct((_N, _D2), bf16),
            jax.ShapeDtypeStruct((_C, 1), f32),
            jax.ShapeDtypeStruct((_C, 1), f32),
            jax.ShapeDtypeStruct((_C, _D2), f32),
            jax.ShapeDtypeStruct((_C, _D2), f32),
        ],
    )(x, W1)

    nm = _B // _MB
    y2, s2p, q2p = pl.pallas_call(
        _mid1_kernel,
        grid=(nm,),
        in_specs=[
            pl.BlockSpec((_MB * _ROWS, _D2), lambda b: (b, 0)),
            pl.BlockSpec((_C, 1), _vec),
            pl.BlockSpec((_C, 1), _vec),
            pl.BlockSpec((1, _C), _vec),
            pl.BlockSpec((1, _C), _vec),
            pl.BlockSpec((_C, _D2), _vec),
            pl.BlockSpec((_C, _D2), _vec),
            pl.BlockSpec((_D2, _C), _vec),
            pl.BlockSpec((1, _D2), _vec),
            pl.BlockSpec((1, _D2), _vec),
            pl.BlockSpec((1, _D2), _vec),
            pl.BlockSpec((_D2, _D2), _vec),
            pl.BlockSpec((1, _D2), _vec),
        ],
        out_specs=[
            pl.BlockSpec((_MB * _ROWS, _D2), lambda b: (b, 0)),
            pl.BlockSpec((1, 1, _D2), lambda b: (b, 0, 0)),
            pl.BlockSpec((1, 1, _D2), lambda b: (b, 0, 0)),
        ],
        out_shape=[
            jax.ShapeDtypeStruct((_N, _D2), bf16),
            jax.ShapeDtypeStruct((nm, 1, _D2), f32),
            jax.ShapeDtypeStruct((nm, 1, _D2), f32),
        ],
    )(u, s0, q0, g0c, b0c, U1, U2, W1, bias1.reshape(1, _D2),
      g1r, b1r, W2, bias2.reshape(1, _D2))

    def mid(y, sp, qp, w, bias):
        nm = _B // _MB
        return pl.pallas_call(
            _mid_kernel,
            grid=(nm,),
            in_specs=[
                pl.BlockSpec((_MB * _ROWS, _D2), lambda b: (b, 0)),
                pl.BlockSpec(sp.shape, _vec3),
                pl.BlockSpec(qp.shape, _vec3),
                pl.BlockSpec((1, _D2), _vec),
                pl.BlockSpec((1, _D2), _vec),
                pl.BlockSpec((_D2, _D2), _vec),
                pl.BlockSpec((1, _D2), _vec),
            ],
            out_specs=[
                pl.BlockSpec((_MB * _ROWS, _D2), lambda b: (b, 0)),
                pl.BlockSpec((1, 1, _D2), lambda b: (b, 0, 0)),
                pl.BlockSpec((1, 1, _D2), lambda b: (b, 0, 0)),
            ],
            out_shape=[
                jax.ShapeDtypeStruct((_N, _D2), bf16),
                jax.ShapeDtypeStruct((nm, 1, _D2), f32),
                jax.ShapeDtypeStruct((nm, 1, _D2), f32),
            ],
        )(y, sp, qp, g1r, b1r, w, bias.reshape(1, _D2))


    y3, s3p, q3p = mid(y2, s2p, q2p, W3, bias3)

    out = pl.pallas_call(
        _tail_kernel,
        grid=(_B // _TB,),
        in_specs=[
            pl.BlockSpec((_TB, _C, _L), lambda b: (b, 0, 0)),
            pl.BlockSpec((_C, 1), _vec),
            pl.BlockSpec((_C, 1), _vec),
            pl.BlockSpec((1, _C), _vec),
            pl.BlockSpec((1, _C), _vec),
            pl.BlockSpec((_D2, _TK * _C), _vec),
            pl.BlockSpec((1, _D2), _vec),
            pl.BlockSpec((_TB * _ROWS, _D2), lambda b: (b, 0)),
            pl.BlockSpec(s3p.shape, _vec3),
            pl.BlockSpec(q3p.shape, _vec3),
            pl.BlockSpec((1, _D2), _vec),
            pl.BlockSpec((1, _D2), _vec),
        ],
        out_specs=pl.BlockSpec((_TB, _D2, _L), lambda b: (b, 0, 0)),
        out_shape=jax.ShapeDtypeStruct((_B, _D2, _L), f32),
    )(x, s0, q0, g0c, b0c, wc, bsk, y3, s3p, q3p, g1r, b1r)

    return out
